# Initial kernel scaffold; baseline (speedup 1.0000x reference)
import jax
import jax.numpy as jnp
from jax.experimental import pallas as pl


def kernel(idx, token_embedding_table):
    raise NotImplementedError("write your pallas kernel here")


# speedup vs baseline: 1.4159x; 1.4159x over previous
import functools

import jax
import jax.numpy as jnp
from jax import lax
from jax.experimental import pallas as pl
from jax.experimental.pallas import tpu as pltpu
from jax.experimental.pallas import tpu_sc as plsc

VOCAB = 1000
BT = 4096 * 20
NC = 2
NS = 16
NW = NC * NS
B_PER_W = BT // NW
CHUNK = 80
N_CHUNKS = B_PER_W // CHUNK


def _gather_body(table_hbm, idx_hbm, out_hbm, idx_v, rows_v, sem):
    wid = lax.axis_index("s") * NC + lax.axis_index("c")
    base = wid * B_PER_W
    pltpu.sync_copy(idx_hbm.at[pl.ds(base, B_PER_W)], idx_v)

    def body(j, carry):
        pltpu.async_copy(
            table_hbm.at[idx_v.at[pl.ds(j * CHUNK, CHUNK)]], rows_v, sem
        ).wait()
        pltpu.sync_copy(rows_v, out_hbm.at[pl.ds(base + j * CHUNK, CHUNK)])
        return carry

    lax.fori_loop(0, N_CHUNKS, body, 0)


@jax.jit
def _run(idx_flat, table):
    mesh = plsc.VectorSubcoreMesh(core_axis_name="c", subcore_axis_name="s")
    return pl.kernel(
        _gather_body,
        out_type=jax.ShapeDtypeStruct((BT, VOCAB), jnp.float32),
        mesh=mesh,
        scratch_types=[
            pltpu.VMEM((B_PER_W,), jnp.int32),
            pltpu.VMEM((CHUNK, VOCAB), jnp.float32),
            pltpu.SemaphoreType.DMA,
        ],
        compiler_params=pltpu.CompilerParams(use_tc_tiling_on_sc=False),
    )(table, idx_flat)


def kernel(idx, token_embedding_table):
    out = _run(idx.reshape(-1), token_embedding_table)
    return out.reshape(idx.shape[0], idx.shape[1], VOCAB)


# speedup vs baseline: 1.4372x; 1.0150x over previous
import functools

import jax
import jax.numpy as jnp
from jax import lax
from jax.experimental import pallas as pl
from jax.experimental.pallas import tpu as pltpu
from jax.experimental.pallas import tpu_sc as plsc

VOCAB = 1000
BT = 4096 * 20
NC = 2
NS = 16
NW = NC * NS
B_PER_W = BT // NW
NB = 4
CHUNK = 16
N_CHUNKS = B_PER_W // CHUNK
GROUPS = N_CHUNKS // NB


def _gather_body(table_hbm, idx_hbm, out_hbm, idx_v, rows0, rows1, rows2,
                 rows3, gs0, gs1, gs2, gs3, ws0, ws1, ws2, ws3):
    rows = (rows0, rows1, rows2, rows3)
    gsem = (gs0, gs1, gs2, gs3)
    wsem = (ws0, ws1, ws2, ws3)

    wid = lax.axis_index("s") * NC + lax.axis_index("c")
    base = wid * B_PER_W
    pltpu.sync_copy(idx_hbm.at[pl.ds(base, B_PER_W)], idx_v)

    def gather(j, b):
        return pltpu.make_async_copy(
            table_hbm.at[idx_v.at[pl.ds(j * CHUNK, CHUNK)]], rows[b], gsem[b]
        )

    def write(j, b):
        return pltpu.make_async_copy(
            rows[b], out_hbm.at[pl.ds(base + j * CHUNK, CHUNK)], wsem[b]
        )

    for b in range(NB - 1):
        gather(b, b).start()

    def group(g, carry):
        for b in range(NB):
            j = g * NB + b
            bn = (b + NB - 1) % NB
            if b == 0:
                @pl.when(g >= 1)
                def _():
                    write(j - 1, bn).wait()
                    gather(j + NB - 1, bn).start()

                @pl.when(g == 0)
                def _():
                    gather(j + NB - 1, bn).start()
            else:
                write(j - 1, bn).wait()

                @pl.when(j + NB - 1 < N_CHUNKS)
                def _():
                    gather(j + NB - 1, bn).start()
            gather(j, b).wait()
            write(j, b).start()
        return carry

    lax.fori_loop(0, GROUPS, group, 0)
    write(N_CHUNKS - 1, (N_CHUNKS - 1) % NB).wait()


@jax.jit
def _run(idx_flat, table):
    mesh = plsc.VectorSubcoreMesh(core_axis_name="c", subcore_axis_name="s")
    return pl.kernel(
        _gather_body,
        out_type=jax.ShapeDtypeStruct((BT, VOCAB), jnp.float32),
        mesh=mesh,
        scratch_types=[
            pltpu.VMEM((B_PER_W,), jnp.int32),
            pltpu.VMEM((CHUNK, VOCAB), jnp.float32),
            pltpu.VMEM((CHUNK, VOCAB), jnp.float32),
            pltpu.VMEM((CHUNK, VOCAB), jnp.float32),
            pltpu.VMEM((CHUNK, VOCAB), jnp.float32),
            pltpu.SemaphoreType.DMA,
            pltpu.SemaphoreType.DMA,
            pltpu.SemaphoreType.DMA,
            pltpu.SemaphoreType.DMA,
            pltpu.SemaphoreType.DMA,
            pltpu.SemaphoreType.DMA,
            pltpu.SemaphoreType.DMA,
            pltpu.SemaphoreType.DMA,
        ],
        compiler_params=pltpu.CompilerParams(use_tc_tiling_on_sc=False),
    )(table, idx_flat)


def kernel(idx, token_embedding_table):
    out = _run(idx.reshape(-1), token_embedding_table)
    return out.reshape(idx.shape[0], idx.shape[1], VOCAB)


# speedup vs baseline: 1.6540x; 1.1509x over previous
import functools

import jax
import jax.numpy as jnp
from jax import lax
from jax.experimental import pallas as pl
from jax.experimental.pallas import tpu as pltpu
from jax.experimental.pallas import tpu_sc as plsc

VOCAB = 1000
BT = 4096 * 20
NC = 2
NS = 16
NW = NC * NS
B_PER_W = BT // NW
NB = 4
CHUNK = 16
N_CHUNKS = B_PER_W // CHUNK
GROUPS = N_CHUNKS // NB


STAGE_ROWS = VOCAB // NS
STAGE_REM = VOCAB - STAGE_ROWS * NS


def _gather_body(table_hbm, idx_hbm, out_hbm, idx_v, rows0, rows1, rows2,
                 rows3, table_sp, gs0, gs1, gs2, gs3, ws0, ws1, ws2, ws3):
    rows = (rows0, rows1, rows2, rows3)
    gsem = (gs0, gs1, gs2, gs3)
    wsem = (ws0, ws1, ws2, ws3)

    sid = lax.axis_index("s")
    wid = sid * NC + lax.axis_index("c")
    base = wid * B_PER_W
    pltpu.sync_copy(idx_hbm.at[pl.ds(base, B_PER_W)], idx_v)

    pltpu.sync_copy(
        table_hbm.at[pl.ds(sid * STAGE_ROWS, STAGE_ROWS)],
        table_sp.at[pl.ds(sid * STAGE_ROWS, STAGE_ROWS)],
    )

    @pl.when(sid == 0)
    def _():
        pltpu.sync_copy(
            table_hbm.at[pl.ds(NS * STAGE_ROWS, STAGE_REM)],
            table_sp.at[pl.ds(NS * STAGE_ROWS, STAGE_REM)],
        )

    plsc.subcore_barrier()

    def gather(j, b):
        return pltpu.make_async_copy(
            table_sp.at[idx_v.at[pl.ds(j * CHUNK, CHUNK)]], rows[b], gsem[b]
        )

    def write(j, b):
        return pltpu.make_async_copy(
            rows[b], out_hbm.at[pl.ds(base + j * CHUNK, CHUNK)], wsem[b]
        )

    for b in range(NB - 1):
        gather(b, b).start()

    def group(g, carry):
        for b in range(NB):
            j = g * NB + b
            bn = (b + NB - 1) % NB
            if b == 0:
                @pl.when(g >= 1)
                def _():
                    write(j - 1, bn).wait()
                    gather(j + NB - 1, bn).start()

                @pl.when(g == 0)
                def _():
                    gather(j + NB - 1, bn).start()
            else:
                write(j - 1, bn).wait()

                @pl.when(j + NB - 1 < N_CHUNKS)
                def _():
                    gather(j + NB - 1, bn).start()
            gather(j, b).wait()
            write(j, b).start()
        return carry

    lax.fori_loop(0, GROUPS, group, 0)
    write(N_CHUNKS - 1, (N_CHUNKS - 1) % NB).wait()


@jax.jit
def _run(idx_flat, table):
    mesh = plsc.VectorSubcoreMesh(core_axis_name="c", subcore_axis_name="s")
    return pl.kernel(
        _gather_body,
        out_type=jax.ShapeDtypeStruct((BT, VOCAB), jnp.float32),
        mesh=mesh,
        scratch_types=[
            pltpu.VMEM((B_PER_W,), jnp.int32),
            pltpu.VMEM((CHUNK, VOCAB), jnp.float32),
            pltpu.VMEM((CHUNK, VOCAB), jnp.float32),
            pltpu.VMEM((CHUNK, VOCAB), jnp.float32),
            pltpu.VMEM((CHUNK, VOCAB), jnp.float32),
            pltpu.VMEM_SHARED((VOCAB, VOCAB), jnp.float32),
            pltpu.SemaphoreType.DMA,
            pltpu.SemaphoreType.DMA,
            pltpu.SemaphoreType.DMA,
            pltpu.SemaphoreType.DMA,
            pltpu.SemaphoreType.DMA,
            pltpu.SemaphoreType.DMA,
            pltpu.SemaphoreType.DMA,
            pltpu.SemaphoreType.DMA,
        ],
        compiler_params=pltpu.CompilerParams(use_tc_tiling_on_sc=False),
    )(table, idx_flat)


def kernel(idx, token_embedding_table):
    out = _run(idx.reshape(-1), token_embedding_table)
    return out.reshape(idx.shape[0], idx.shape[1], VOCAB)
